# baseline (speedup 1.0000x reference)
import jax
import jax.numpy as jnp
from jax import lax
from jax.experimental import pallas as pl
from jax.experimental.pallas import tpu as pltpu
from jax.experimental.pallas import tpu_sc as plsc

B, S, D = 4096, 200, 128
BS = B * S
NC, NS, L = 2, 16, 16
NW = NC * NS
PW = BS // NW
CH = 64
NIT = PW // CH


def _body(xf, t0, t1, t2, t3, out, xbuf, ibuf, rbuf, s0, s1, s2, s3, gsem):
    sid = lax.axis_index("s")
    wid = sid * NC + lax.axis_index("c")
    base = wid * PW
    tabs = (t0, t1, t2, t3)
    shtabs = (s0, s1, s2, s3)

    bounds = (0, 256, 512, 768, 1000)
    for t in range(4):
        for q in range(4):
            @pl.when(sid == q * 4 + t)
            def _(t=t, q=q):
                lo, hi = bounds[q], bounds[q + 1]
                pltpu.sync_copy(
                    tabs[t].at[pl.ds(lo, hi - lo)],
                    shtabs[t].at[pl.ds(lo, hi - lo)],
                )
    plsc.subcore_barrier()

    def stage_a(i, p):
        cbase = base + i * CH
        pltpu.sync_copy(xf.at[pl.ds(cbase * 4, CH * 4)], xbuf.at[p])

        def deint(k, _):
            lanes = lax.iota(jnp.int32, L) * 4 + k * (4 * L)
            pvec = jnp.broadcast_to(p, (L,)).astype(jnp.int32)
            for t in range(4):
                v = plsc.load_gather(xbuf, [pvec, lanes + t])
                ibuf[p, t, pl.ds(k * L, L)] = v
            return 0

        lax.fori_loop(0, CH // L, deint, 0)
        for t in range(4):
            pltpu.async_copy(
                shtabs[t].at[ibuf.at[p, t]], rbuf.at[p, t], gsem.at[p]
            )

    def stage_b(i, p):
        for t in range(4):
            pltpu.make_async_copy(
                shtabs[t].at[ibuf.at[p, t]], rbuf.at[p, t], gsem.at[p]
            ).wait()

        def accum(j, _):
            for l in range(D // L):
                s = pl.ds(l * L, L)
                v = rbuf[p, 1, j, s] + rbuf[p, 2, j, s] + rbuf[p, 3, j, s]
                plsc.addupdate(rbuf.at[p, 0, j, s], v)
            return 0

        lax.fori_loop(0, CH, accum, 0)
        pltpu.sync_copy(rbuf.at[p, 0], out.at[pl.ds(base + i * CH, CH)])

    stage_a(0, 0)

    def step(i, _):
        p = lax.rem(i, 2)

        @pl.when(i + 1 < NIT)
        def _():
            stage_a(i + 1, 1 - p)

        stage_b(i, p)
        return 0

    lax.fori_loop(0, NIT, step, 0)


@jax.jit
def kernel(x, onset_table, medial_table, nucleus_table, coda_table):
    xf = x.reshape(-1)
    mesh = plsc.VectorSubcoreMesh(core_axis_name="c", subcore_axis_name="s")
    kfn = pl.kernel(
        _body,
        out_type=jax.ShapeDtypeStruct((BS, D), jnp.float32),
        mesh=mesh,
        compiler_params=pltpu.CompilerParams(needs_layout_passes=False),
        scratch_types=[
            pltpu.VMEM((2, CH * 4), jnp.int32),
            pltpu.VMEM((2, 4, CH), jnp.int32),
            pltpu.VMEM((2, 4, CH, D), jnp.float32),
            pltpu.VMEM_SHARED((1000, D), jnp.float32),
            pltpu.VMEM_SHARED((1000, D), jnp.float32),
            pltpu.VMEM_SHARED((1000, D), jnp.float32),
            pltpu.VMEM_SHARED((1000, D), jnp.float32),
            pltpu.SemaphoreType.DMA((2,)),
        ],
    )
    out = kfn(xf, onset_table, medial_table, nucleus_table, coda_table)
    return out.reshape(B, S, D)


# speedup vs baseline: 1.6683x; 1.6683x over previous
import jax
import jax.numpy as jnp
from jax import lax
from jax.experimental import pallas as pl
from jax.experimental.pallas import tpu as pltpu
from jax.experimental.pallas import tpu_sc as plsc

B, S, D = 4096, 200, 128
BS = B * S
NC, NS, L = 2, 16, 16
NW = NC * NS
PW = BS // NW
CH = 64
NIT = PW // CH


def _body(xf, t0, t1, t2, t3, out, xbuf, ibuf, rbuf,
          s0, s1, s2, s3, xsem, gsem, osem):
    sid = lax.axis_index("s")
    wid = sid * NC + lax.axis_index("c")
    base = wid * PW
    tabs = (t0, t1, t2, t3)
    shtabs = (s0, s1, s2, s3)

    bounds = (0, 256, 512, 768, 1000)
    for t in range(4):
        for q in range(4):
            @pl.when(sid == q * 4 + t)
            def _(t=t, q=q):
                lo, hi = bounds[q], bounds[q + 1]
                pltpu.sync_copy(
                    tabs[t].at[pl.ds(lo, hi - lo)],
                    shtabs[t].at[pl.ds(lo, hi - lo)],
                )
    plsc.subcore_barrier()

    def fire_x(j, p):
        pltpu.async_copy(
            xf.at[pl.ds((base + j * CH) * 4, CH * 4)], xbuf.at[p], xsem.at[p]
        )

    def wait_x(p):
        pltpu.make_async_copy(
            xf.at[pl.ds(base * 4, CH * 4)], xbuf.at[p], xsem.at[p]
        ).wait()

    def fire_gathers(p):
        for t in range(4):
            pltpu.async_copy(
                shtabs[t].at[ibuf.at[p, t]], rbuf.at[p, t], gsem.at[p]
            )

    def wait_gathers(p):
        for t in range(4):
            pltpu.make_async_copy(
                shtabs[t].at[ibuf.at[p, t]], rbuf.at[p, t], gsem.at[p]
            ).wait()

    def fire_out(j, p):
        pltpu.async_copy(
            rbuf.at[p, 0], out.at[pl.ds(base + j * CH, CH)], osem.at[p]
        )

    def wait_out(p):
        pltpu.make_async_copy(
            rbuf.at[p, 0], out.at[pl.ds(base, CH)], osem.at[p]
        ).wait()

    def deint_fire(j, p):
        wait_x(p)

        @plsc.parallel_loop(0, CH // L, unroll=2)
        def _(k):
            lanes = lax.iota(jnp.int32, L) * 4 + k * (4 * L)
            pvec = jnp.broadcast_to(p, (L,)).astype(jnp.int32)
            for t in range(4):
                v = plsc.load_gather(xbuf, [pvec, lanes + t])
                ibuf[p, t, pl.ds(k * L, L)] = v

        @pl.when(j >= 2)
        def _():
            wait_out(p)

        fire_gathers(p)

    def accum_out(j, p):
        wait_gathers(p)

        @plsc.parallel_loop(0, CH, unroll=2)
        def _(r):
            for l in range(D // L):
                s = pl.ds(l * L, L)
                v = rbuf[p, 1, r, s] + rbuf[p, 2, r, s] + rbuf[p, 3, r, s]
                plsc.addupdate(rbuf.at[p, 0, r, s], v)

        fire_out(j, p)

    fire_x(0, 0)
    fire_x(1, 1)
    deint_fire(0, 0)

    def step(i, _):
        p = lax.rem(i, 2)

        @pl.when(i + 2 < NIT)
        def _():
            fire_x(i + 2, p)

        @pl.when(i + 1 < NIT)
        def _():
            deint_fire(i + 1, 1 - p)

        accum_out(i, p)
        return 0

    lax.fori_loop(0, NIT, step, 0)
    wait_out(0)
    wait_out(1)


@jax.jit
def kernel(x, onset_table, medial_table, nucleus_table, coda_table):
    xf = x.reshape(-1)
    mesh = plsc.VectorSubcoreMesh(core_axis_name="c", subcore_axis_name="s")
    kfn = pl.kernel(
        _body,
        out_type=jax.ShapeDtypeStruct((BS, D), jnp.float32),
        mesh=mesh,
        compiler_params=pltpu.CompilerParams(needs_layout_passes=False),
        scratch_types=[
            pltpu.VMEM((2, CH * 4), jnp.int32),
            pltpu.VMEM((2, 4, CH), jnp.int32),
            pltpu.VMEM((2, 4, CH, D), jnp.float32),
            pltpu.VMEM_SHARED((1000, D), jnp.float32),
            pltpu.VMEM_SHARED((1000, D), jnp.float32),
            pltpu.VMEM_SHARED((1000, D), jnp.float32),
            pltpu.VMEM_SHARED((1000, D), jnp.float32),
            pltpu.SemaphoreType.DMA((2,)),
            pltpu.SemaphoreType.DMA((2,)),
            pltpu.SemaphoreType.DMA((2,)),
        ],
    )
    out = kfn(xf, onset_table, medial_table, nucleus_table, coda_table)
    return out.reshape(B, S, D)


# speedup vs baseline: 1.6709x; 1.0016x over previous
import jax
import jax.numpy as jnp
from jax import lax
from jax.experimental import pallas as pl
from jax.experimental.pallas import tpu as pltpu
from jax.experimental.pallas import tpu_sc as plsc

B, S, D = 4096, 200, 128
BS = B * S
NC, NS, L = 2, 16, 16
NW = NC * NS
PW = BS // NW
CH = 64
NIT = PW // CH


def _body(xf, t0, t1, t2, t3, out, xbuf, ibuf, rbuf,
          s0, s1, s2, s3, xsem, gsem, osem):
    sid = lax.axis_index("s")
    wid = sid * NC + lax.axis_index("c")
    base = wid * PW
    tabs = (t0, t1, t2, t3)
    shtabs = (s0, s1, s2, s3)

    bounds = (0, 256, 512, 768, 1000)
    for t in range(4):
        for q in range(4):
            @pl.when(sid == q * 4 + t)
            def _(t=t, q=q):
                lo, hi = bounds[q], bounds[q + 1]
                pltpu.sync_copy(
                    tabs[t].at[pl.ds(lo, hi - lo)],
                    shtabs[t].at[pl.ds(lo, hi - lo)],
                )
    plsc.subcore_barrier()

    def fire_x(j, p):
        pltpu.async_copy(
            xf.at[pl.ds((base + j * CH) * 4, CH * 4)], xbuf.at[p], xsem.at[p]
        )

    def wait_x(p):
        pltpu.make_async_copy(
            xf.at[pl.ds(base * 4, CH * 4)], xbuf.at[p], xsem.at[p]
        ).wait()

    def fire_gathers(p):
        for t in range(4):
            pltpu.async_copy(
                shtabs[t].at[ibuf.at[p, t]], rbuf.at[p, t], gsem.at[p]
            )

    def wait_gathers(p):
        for t in range(4):
            pltpu.make_async_copy(
                shtabs[t].at[ibuf.at[p, t]], rbuf.at[p, t], gsem.at[p]
            ).wait()

    def fire_out(j, p):
        pltpu.async_copy(
            rbuf.at[p, 0], out.at[pl.ds(base + j * CH, CH)], osem.at[p]
        )

    def wait_out(p):
        pltpu.make_async_copy(
            rbuf.at[p, 0], out.at[pl.ds(base, CH)], osem.at[p]
        ).wait()

    def deint_fire(j, p):
        wait_x(p)

        @plsc.parallel_loop(0, CH // L, unroll=2)
        def _(k):
            lanes = lax.iota(jnp.int32, L) * 4 + k * (4 * L)
            pvec = jnp.broadcast_to(p, (L,)).astype(jnp.int32)
            for t in range(4):
                v = plsc.load_gather(xbuf, [pvec, lanes + t])
                ibuf[p, t, pl.ds(k * L, L)] = v

        @pl.when(j >= 2)
        def _():
            wait_out(p)

        fire_gathers(p)

    def accum_out(j, p):
        wait_gathers(p)

        @plsc.parallel_loop(0, CH, unroll=4)
        def _(r):
            for l in range(D // L):
                s = pl.ds(l * L, L)
                v = rbuf[p, 1, r, s] + rbuf[p, 2, r, s] + rbuf[p, 3, r, s]
                plsc.addupdate(rbuf.at[p, 0, r, s], v)

        fire_out(j, p)

    fire_x(0, 0)
    fire_x(1, 1)
    deint_fire(0, 0)

    def step(i, _):
        p = lax.rem(i, 2)

        @pl.when(i + 2 < NIT)
        def _():
            fire_x(i + 2, p)

        @pl.when(i + 1 < NIT)
        def _():
            deint_fire(i + 1, 1 - p)

        accum_out(i, p)
        return 0

    lax.fori_loop(0, NIT, step, 0)
    wait_out(0)
    wait_out(1)


@jax.jit
def kernel(x, onset_table, medial_table, nucleus_table, coda_table):
    xf = x.reshape(-1)
    mesh = plsc.VectorSubcoreMesh(core_axis_name="c", subcore_axis_name="s")
    kfn = pl.kernel(
        _body,
        out_type=jax.ShapeDtypeStruct((BS, D), jnp.float32),
        mesh=mesh,
        compiler_params=pltpu.CompilerParams(needs_layout_passes=False),
        scratch_types=[
            pltpu.VMEM((2, CH * 4), jnp.int32),
            pltpu.VMEM((2, 4, CH), jnp.int32),
            pltpu.VMEM((2, 4, CH, D), jnp.float32),
            pltpu.VMEM_SHARED((1000, D), jnp.float32),
            pltpu.VMEM_SHARED((1000, D), jnp.float32),
            pltpu.VMEM_SHARED((1000, D), jnp.float32),
            pltpu.VMEM_SHARED((1000, D), jnp.float32),
            pltpu.SemaphoreType.DMA((2,)),
            pltpu.SemaphoreType.DMA((2,)),
            pltpu.SemaphoreType.DMA((2,)),
        ],
    )
    out = kfn(xf, onset_table, medial_table, nucleus_table, coda_table)
    return out.reshape(B, S, D)
